# baseline (speedup 1.0000x reference)
import functools

import jax
import jax.numpy as jnp
from jax import lax
from jax.experimental import pallas as pl
from jax.experimental.pallas import tpu as pltpu
from jax.experimental.pallas import tpu_sc as plsc

B, C, N, M = 8, 64, 50000, 16384
NC, NS, L = 2, 16, 16
NW = NC * NS
WPB = NW // B
CPW = C // WPB
OCHUNK = 2048
NCHUNK = M // OCHUNK


def _gather_body(features, idx, out, idx_v, feat_a, feat_b, out_a, out_b,
                 fsem_a, fsem_b, osem_a, osem_b):
    wid = lax.axis_index("s") * NC + lax.axis_index("c")
    b = wid // WPB
    c0 = (wid % WPB) * CPW

    feat_bufs = (feat_a, feat_b)
    fsems = (fsem_a, fsem_b)
    out_bufs = (out_a, out_b)
    osems = (osem_a, osem_b)

    pltpu.make_async_copy(features.at[b, c0], feat_a, fsem_a).start()
    pltpu.sync_copy(idx.at[b], idx_v)

    pending = [None, None]
    for k in range(CPW):
        fb = feat_bufs[k % 2]
        if k + 1 < CPW:
            pltpu.make_async_copy(
                features.at[b, c0 + k + 1],
                feat_bufs[(k + 1) % 2], fsems[(k + 1) % 2]).start()
        pltpu.make_async_copy(
            features.at[b, c0 + k], fb, fsems[k % 2]).wait()

        for h in range(NCHUNK):
            oi = (k * NCHUNK + h) % 2
            ob = out_bufs[oi]
            if pending[oi] is not None:
                pending[oi].wait()

            base = h * OCHUNK

            @plsc.parallel_loop(0, OCHUNK, step=L, unroll=8)
            def _gather_chunk(i, fb=fb, ob=ob, base=base):
                iv = idx_v[pl.ds(base + i, L)]
                ob[pl.ds(i, L)] = plsc.load_gather(fb, [iv])

            cp = pltpu.make_async_copy(
                ob, out.at[b, c0 + k, pl.ds(base, OCHUNK)], osems[oi])
            cp.start()
            pending[oi] = cp

    for cp in pending:
        if cp is not None:
            cp.wait()


@jax.jit
def kernel(features, idx):
    mesh = plsc.VectorSubcoreMesh(core_axis_name="c", subcore_axis_name="s")
    run = functools.partial(
        pl.kernel,
        out_type=jax.ShapeDtypeStruct((B, C, M), jnp.float32),
        mesh=mesh,
        compiler_params=pltpu.CompilerParams(needs_layout_passes=False),
        scratch_types=[
            pltpu.VMEM((M,), jnp.int32),
            pltpu.VMEM((N,), jnp.float32),
            pltpu.VMEM((N,), jnp.float32),
            pltpu.VMEM((OCHUNK,), jnp.float32),
            pltpu.VMEM((OCHUNK,), jnp.float32),
            pltpu.SemaphoreType.DMA,
            pltpu.SemaphoreType.DMA,
            pltpu.SemaphoreType.DMA,
            pltpu.SemaphoreType.DMA,
        ],
    )(_gather_body)
    return run(features, idx.astype(jnp.int32))


# speedup vs baseline: 1.0648x; 1.0648x over previous
import functools

import jax
import jax.numpy as jnp
from jax import lax
from jax.experimental import pallas as pl
from jax.experimental.pallas import tpu as pltpu
from jax.experimental.pallas import tpu_sc as plsc

B, C, N, M = 8, 64, 50000, 16384
NC, NS, L = 2, 16, 16
NW = NC * NS
WPB = NW // B
CPW = C // WPB
OCHUNK = 4096
NCHUNK = M // OCHUNK


def _gather_body(features, idx, out, idx_v, feat_a, feat_b, out_a, out_b,
                 fsem_a, fsem_b, osem_a, osem_b):
    wid = lax.axis_index("s") * NC + lax.axis_index("c")
    b = wid // WPB
    c0 = (wid % WPB) * CPW

    feat_bufs = (feat_a, feat_b)
    fsems = (fsem_a, fsem_b)
    out_bufs = (out_a, out_b)
    osems = (osem_a, osem_b)

    pltpu.make_async_copy(features.at[b, c0], feat_a, fsem_a).start()
    pltpu.sync_copy(idx.at[b], idx_v)

    pending = [None, None]
    for k in range(CPW):
        fb = feat_bufs[k % 2]
        if k + 1 < CPW:
            pltpu.make_async_copy(
                features.at[b, c0 + k + 1],
                feat_bufs[(k + 1) % 2], fsems[(k + 1) % 2]).start()
        pltpu.make_async_copy(
            features.at[b, c0 + k], fb, fsems[k % 2]).wait()

        for h in range(NCHUNK):
            oi = (k * NCHUNK + h) % 2
            ob = out_bufs[oi]
            if pending[oi] is not None:
                pending[oi].wait()

            base = h * OCHUNK

            @plsc.parallel_loop(0, OCHUNK, step=L, unroll=8)
            def _gather_chunk(i, fb=fb, ob=ob, base=base):
                iv = idx_v[pl.ds(base + i, L)]
                ob[pl.ds(i, L)] = plsc.load_gather(fb, [iv])

            cp = pltpu.make_async_copy(
                ob, out.at[b, c0 + k, pl.ds(base, OCHUNK)], osems[oi])
            cp.start()
            pending[oi] = cp

    for cp in pending:
        if cp is not None:
            cp.wait()


@jax.jit
def kernel(features, idx):
    mesh = plsc.VectorSubcoreMesh(core_axis_name="c", subcore_axis_name="s")
    run = functools.partial(
        pl.kernel,
        out_type=jax.ShapeDtypeStruct((B, C, M), jnp.float32),
        mesh=mesh,
        compiler_params=pltpu.CompilerParams(needs_layout_passes=False),
        scratch_types=[
            pltpu.VMEM((M,), jnp.int32),
            pltpu.VMEM((N,), jnp.float32),
            pltpu.VMEM((N,), jnp.float32),
            pltpu.VMEM((OCHUNK,), jnp.float32),
            pltpu.VMEM((OCHUNK,), jnp.float32),
            pltpu.SemaphoreType.DMA,
            pltpu.SemaphoreType.DMA,
            pltpu.SemaphoreType.DMA,
            pltpu.SemaphoreType.DMA,
        ],
    )(_gather_body)
    return run(features, idx.astype(jnp.int32))


# speedup vs baseline: 1.0661x; 1.0012x over previous
import functools

import jax
import jax.numpy as jnp
from jax import lax
from jax.experimental import pallas as pl
from jax.experimental.pallas import tpu as pltpu
from jax.experimental.pallas import tpu_sc as plsc

B, C, N, M = 8, 64, 50000, 16384
NC, NS, L = 2, 16, 16
NW = NC * NS
WPB = NW // B
CPW = C // WPB
OCHUNK = 4096
NCHUNK = M // OCHUNK


def _gather_body(features, idx, out, idx_v, feat_a, feat_b, out_a, out_b,
                 out_c, fsem_a, fsem_b, osem_a, osem_b, osem_c):
    wid = lax.axis_index("s") * NC + lax.axis_index("c")
    b = wid // WPB
    c0 = (wid % WPB) * CPW

    feat_bufs = (feat_a, feat_b)
    fsems = (fsem_a, fsem_b)
    out_bufs = (out_a, out_b, out_c)
    osems = (osem_a, osem_b, osem_c)

    pltpu.make_async_copy(features.at[b, c0], feat_a, fsem_a).start()
    pltpu.sync_copy(idx.at[b], idx_v)

    pending = [None, None, None]
    for k in range(CPW):
        fb = feat_bufs[k % 2]
        if k + 1 < CPW:
            pltpu.make_async_copy(
                features.at[b, c0 + k + 1],
                feat_bufs[(k + 1) % 2], fsems[(k + 1) % 2]).start()
        pltpu.make_async_copy(
            features.at[b, c0 + k], fb, fsems[k % 2]).wait()

        for h in range(NCHUNK):
            oi = (k * NCHUNK + h) % 3
            ob = out_bufs[oi]
            if pending[oi] is not None:
                pending[oi].wait()

            base = h * OCHUNK

            @plsc.parallel_loop(0, OCHUNK, step=L, unroll=8)
            def _gather_chunk(i, fb=fb, ob=ob, base=base):
                iv = idx_v[pl.ds(base + i, L)]
                ob[pl.ds(i, L)] = plsc.load_gather(fb, [iv])

            cp = pltpu.make_async_copy(
                ob, out.at[b, c0 + k, pl.ds(base, OCHUNK)], osems[oi])
            cp.start()
            pending[oi] = cp

    for cp in pending:
        if cp is not None:
            cp.wait()


@jax.jit
def kernel(features, idx):
    mesh = plsc.VectorSubcoreMesh(core_axis_name="c", subcore_axis_name="s")
    run = functools.partial(
        pl.kernel,
        out_type=jax.ShapeDtypeStruct((B, C, M), jnp.float32),
        mesh=mesh,
        compiler_params=pltpu.CompilerParams(needs_layout_passes=False),
        scratch_types=[
            pltpu.VMEM((M,), jnp.int32),
            pltpu.VMEM((N,), jnp.float32),
            pltpu.VMEM((N,), jnp.float32),
            pltpu.VMEM((OCHUNK,), jnp.float32),
            pltpu.VMEM((OCHUNK,), jnp.float32),
            pltpu.VMEM((OCHUNK,), jnp.float32),
            pltpu.SemaphoreType.DMA,
            pltpu.SemaphoreType.DMA,
            pltpu.SemaphoreType.DMA,
            pltpu.SemaphoreType.DMA,
            pltpu.SemaphoreType.DMA,
        ],
    )(_gather_body)
    return run(features, idx.astype(jnp.int32))
